# baseline (speedup 1.0000x reference)
import functools

import jax
import jax.numpy as jnp
from jax import lax
from jax.experimental import pallas as pl
from jax.experimental.pallas import tpu as pltpu
from jax.experimental.pallas import tpu_sc as plsc

_K = 256
_M = 128
_N = 32768
_L = 16
_NW = 32
_ROWS_PER_W = _M // _NW
_NV_ROW = _N // _L
_PIV = 0x40000000
_MININT = -2147483648


def _splat(val, dtype=jnp.int32):
    return jnp.full((_L,), val, dtype=dtype)


def _sc_body(x_hbm, out_hbm, row_v, cand_v):
    wid = lax.axis_index("s") * 2 + lax.axis_index("c")

    lanes = lax.iota(jnp.int32, _L)
    one_i = _splat(1)
    zero_i = _splat(0)
    min_i = _splat(_MININT)

    def _key(b):
        return jnp.where(b >= zero_i, b, min_i - b)

    def compact_below(flip):
        piv = _splat(_PIV)
        flip_v = jnp.full((_L,), flip, jnp.bool_)

        def body(i, carry):
            woff, acc = carry
            b = row_v[pl.ds(i * _L, _L)]
            s = _key(b)
            keep = (s >= piv) != flip_v
            ki = jnp.where(keep, one_i, zero_i)
            pos = woff + plsc.cumsum(ki) - one_i
            plsc.store_scatter(cand_v, [pos], s, mask=keep)
            pc = plsc.all_reduce_population_count(keep)
            return woff + pc, acc + ki

        woff, acc = lax.fori_loop(0, _NV_ROW, body, (zero_i, zero_i),
                                  unroll=4)
        return jnp.sum(acc)

    def row_body(r, _):
        row = wid * _ROWS_PER_W + r
        pltpu.sync_copy(x_hbm.at[row], row_v)

        cnt1 = compact_below(False)
        fast = cnt1 >= _K

        @pl.when(jnp.logical_not(fast))
        def _fallback():
            compact_below(True)

        ln0 = jnp.where(fast, cnt1, _N - cnt1)
        need0 = jnp.where(fast, _K, _K - cnt1)
        prefix0 = jnp.where(fast, _splat(_PIV), zero_i)
        bit0 = jnp.where(fast, 29, 31)

        def bit_cond(st):
            bit, _, need, ln = st
            return jnp.logical_and(bit >= 0, ln != need)

        def bit_body(st):
            bit, prefix, need, ln = st
            m = lax.shift_left(one_i, jnp.full((_L,), bit, jnp.int32))
            flip_v = jnp.full((_L,), bit == 31, jnp.bool_)
            nv = (ln + (_L - 1)) // _L
            ln_s = jnp.full((_L,), ln, jnp.int32)

            def count_body(i, acc):
                v = cand_v[pl.ds(i * _L, _L)]
                valid = (lanes + i * _L) < ln_s
                hi = ((v & m) != zero_i) != flip_v
                return acc + jnp.where(jnp.logical_and(valid, hi),
                                       one_i, zero_i)

            c1 = jnp.sum(lax.fori_loop(0, nv, count_body, zero_i))
            takehi = c1 >= need
            takehi_v = jnp.full((_L,), takehi, jnp.bool_)
            setbit_v = takehi_v != flip_v

            def comp_body(i, woff):
                v = cand_v[pl.ds(i * _L, _L)]
                valid = (lanes + i * _L) < ln_s
                hi = ((v & m) != zero_i) != flip_v
                keep = jnp.logical_and(valid, hi == takehi_v)
                ki = jnp.where(keep, one_i, zero_i)
                pos = woff + plsc.cumsum(ki) - one_i
                plsc.store_scatter(cand_v, [pos], v, mask=keep)
                return woff + plsc.all_reduce_population_count(keep)

            lax.fori_loop(0, nv, comp_body, zero_i)
            new_ln = jnp.where(takehi, c1, ln - c1)
            new_need = jnp.where(takehi, need, need - c1)
            new_prefix = jnp.where(setbit_v, prefix | m, prefix)
            return bit - 1, new_prefix, new_need, new_ln

        _, prefix, _, _ = lax.while_loop(bit_cond, bit_body,
                                         (bit0, prefix0, need0, ln0))

        def mask_body(i, _c):
            b = row_v[pl.ds(i * _L, _L)]
            keep = _key(b) >= prefix
            row_v[pl.ds(i * _L, _L)] = jnp.where(keep, b, zero_i)
            return _c

        lax.fori_loop(0, _NV_ROW, mask_body, 0, unroll=8)
        pltpu.sync_copy(row_v, out_hbm.at[row])
        return 0

    lax.fori_loop(0, _ROWS_PER_W, row_body, 0)


def kernel(x):
    mesh = plsc.VectorSubcoreMesh(core_axis_name="c", subcore_axis_name="s")
    f = functools.partial(
        pl.kernel,
        mesh=mesh,
        out_type=jax.ShapeDtypeStruct((_M, _N), jnp.int32),
        scratch_types=[
            pltpu.VMEM((_N,), jnp.int32),
            pltpu.VMEM((_N,), jnp.int32),
        ],
        compiler_params=pltpu.CompilerParams(needs_layout_passes=False),
    )(_sc_body)
    xb = lax.bitcast_convert_type(x, jnp.int32)
    return lax.bitcast_convert_type(f(xb), jnp.float32)


# speedup vs baseline: 1.3586x; 1.3586x over previous
import functools

import jax
import jax.numpy as jnp
from jax import lax
from jax.experimental import pallas as pl
from jax.experimental.pallas import tpu as pltpu
from jax.experimental.pallas import tpu_sc as plsc

_K = 256
_M = 128
_N = 32768
_L = 16
_NW = 32
_ROWS_PER_W = _M // _NW
_NV_ROW = _N // _L
_PIV = 0x40000000
_MININT = -2147483648


def _splat(val, dtype=jnp.int32):
    return jnp.full((_L,), val, dtype=dtype)


def _sc_body(x_hbm, out_hbm, row_v, cand_v):
    wid = lax.axis_index("s") * 2 + lax.axis_index("c")

    lanes = lax.iota(jnp.int32, _L)
    one_i = _splat(1)
    zero_i = _splat(0)
    min_i = _splat(_MININT)
    sixteen = _splat(_L)
    piv = _splat(_PIV)

    def compact_fast():
        def body(i, posv):
            b = row_v[pl.ds(i * _L, _L)]
            keep = b >= piv
            plsc.store_scatter(cand_v, [posv], b, mask=keep)
            return posv + jnp.where(keep, sixteen, zero_i)

        return lax.fori_loop(0, _NV_ROW, body, lanes, unroll=8)

    def compact_slow():
        def body(i, posv):
            b = row_v[pl.ds(i * _L, _L)]
            s = jnp.where(b >= zero_i, b, min_i - b)
            keep = b < piv
            plsc.store_scatter(cand_v, [posv], s, mask=keep)
            return posv + jnp.where(keep, sixteen, zero_i)

        return lax.fori_loop(0, _NV_ROW, body, lanes, unroll=8)

    def row_body(r, _):
        row = wid * _ROWS_PER_W + r
        pltpu.sync_copy(x_hbm.at[row], row_v)

        endv1 = compact_fast()
        cnt1 = jnp.sum(jnp.right_shift(endv1 - lanes, 4))
        fast = cnt1 >= _K

        endv = lax.cond(fast, lambda: endv1, compact_slow)
        ln0 = jnp.where(fast, cnt1, _N - cnt1)
        need0 = jnp.where(fast, _K, _K - cnt1)
        prefix0 = jnp.where(fast, piv, zero_i)
        bit0 = jnp.where(fast, 29, 31)

        def bit_cond(st):
            bit, _, need, ln, _ = st
            return jnp.logical_and(bit >= 0, ln != need)

        def bit_body(st):
            bit, prefix, need, ln, endv = st
            m = lax.shift_left(one_i, jnp.full((_L,), bit, jnp.int32))
            flip_v = jnp.full((_L,), bit == 31, jnp.bool_)
            maxc = jnp.max(jnp.right_shift(endv - lanes, 4))

            def count_body(i, carry):
                acc, readv = carry
                v = cand_v[pl.ds(i * _L, _L)]
                valid = readv < endv
                hi = ((v & m) != zero_i) != flip_v
                acc = acc + jnp.where(jnp.logical_and(valid, hi),
                                      one_i, zero_i)
                return acc, readv + sixteen

            acc, _ = lax.fori_loop(0, maxc, count_body, (zero_i, lanes))
            c1 = jnp.sum(acc)
            takehi = c1 >= need
            takehi_v = jnp.full((_L,), takehi, jnp.bool_)
            setbit_v = takehi_v != flip_v

            def comp_body(i, carry):
                posv, readv = carry
                v = cand_v[pl.ds(i * _L, _L)]
                valid = readv < endv
                hi = ((v & m) != zero_i) != flip_v
                keep = jnp.logical_and(valid, hi == takehi_v)
                plsc.store_scatter(cand_v, [posv], v, mask=keep)
                return posv + jnp.where(keep, sixteen, zero_i), readv + sixteen

            (new_endv, _) = lax.fori_loop(0, maxc, comp_body, (lanes, lanes))
            new_ln = jnp.where(takehi, c1, ln - c1)
            new_need = jnp.where(takehi, need, need - c1)
            new_prefix = jnp.where(setbit_v, prefix | m, prefix)
            return bit - 1, new_prefix, new_need, new_ln, new_endv

        _, prefix, _, _, _ = lax.while_loop(
            bit_cond, bit_body, (bit0, prefix0, need0, ln0, endv))

        tge0 = prefix >= zero_i
        lo_pos = jnp.where(tge0, prefix, zero_i)
        hi_neg = jnp.where(tge0, min_i, min_i - prefix)

        def mask_body(i, _c):
            b = row_v[pl.ds(i * _L, _L)]
            keep = jnp.logical_or(b >= lo_pos, b <= hi_neg)
            row_v[pl.ds(i * _L, _L)] = jnp.where(keep, b, zero_i)
            return _c

        lax.fori_loop(0, _NV_ROW, mask_body, 0, unroll=8)
        pltpu.sync_copy(row_v, out_hbm.at[row])
        return 0

    lax.fori_loop(0, _ROWS_PER_W, row_body, 0)


def kernel(x):
    mesh = plsc.VectorSubcoreMesh(core_axis_name="c", subcore_axis_name="s")
    f = functools.partial(
        pl.kernel,
        mesh=mesh,
        out_type=jax.ShapeDtypeStruct((_M, _N), jnp.int32),
        scratch_types=[
            pltpu.VMEM((_N,), jnp.int32),
            pltpu.VMEM((_N,), jnp.int32),
        ],
        compiler_params=pltpu.CompilerParams(needs_layout_passes=False),
    )(_sc_body)
    xb = lax.bitcast_convert_type(x, jnp.int32)
    return lax.bitcast_convert_type(f(xb), jnp.float32)


# speedup vs baseline: 1.6813x; 1.2375x over previous
import functools

import jax
import jax.numpy as jnp
from jax import lax
from jax.experimental import pallas as pl
from jax.experimental.pallas import tpu as pltpu
from jax.experimental.pallas import tpu_sc as plsc

_K = 256
_M = 128
_N = 32768
_L = 16
_NW = 32
_ROWS_PER_W = _M // _NW
_NV_ROW = _N // _L
_MININT = -2147483648


def _splat(val, dtype=jnp.int32):
    return jnp.full((_L,), val, dtype=dtype)


def _sc_body(x_hbm, out_hbm, row_v, cand_v):
    wid = lax.axis_index("s") * 2 + lax.axis_index("c")

    lanes = lax.iota(jnp.int32, _L)
    one_i = _splat(1)
    zero_i = _splat(0)
    min_i = _splat(_MININT)
    sixteen = _splat(_L)
    pivf = _splat(2.0, jnp.float32)
    fifteen = _splat(15)

    def compact_fast():
        @plsc.parallel_loop(0, _NV_ROW, unroll=8, carry=lanes)
        def posv_out(i, posv):
            xv = row_v[pl.ds(i * _L, _L)]
            keep = xv >= pivf
            plsc.store_scatter(cand_v, [posv], plsc.bitcast(xv, jnp.int32),
                               mask=keep)
            return posv + jnp.where(keep, sixteen, zero_i)

        return posv_out

    def compact_slow():
        @plsc.parallel_loop(0, _NV_ROW, unroll=8, carry=lanes)
        def posv_out(i, posv):
            xv = row_v[pl.ds(i * _L, _L)]
            b = plsc.bitcast(xv, jnp.int32)
            s = jnp.where(b >= zero_i, b, min_i - b)
            keep = xv < pivf
            plsc.store_scatter(cand_v, [posv], s ^ min_i, mask=keep)
            return posv + jnp.where(keep, sixteen, zero_i)

        return posv_out

    def row_body(r, _):
        row = wid * _ROWS_PER_W + r
        with jax.named_scope("dma_in"):
            pltpu.sync_copy(x_hbm.at[row], row_v)

        with jax.named_scope("phase_a"):
            endv1 = compact_fast()
            cnt1 = jnp.sum(jnp.right_shift(endv1 - lanes, 4))
            fast = cnt1 >= _K
            endv0 = lax.cond(fast, lambda: endv1, compact_slow)

        fast_v = jnp.full((_L,), fast, jnp.bool_)
        maxc0 = jnp.right_shift(jnp.max(endv0 - lanes), 4)
        n_iter = jnp.where(fast, 30, 32)
        m0 = jnp.where(fast_v, _splat(0x20000000), min_i)
        prefix0 = jnp.where(fast_v, _splat(0x40000000), zero_i)
        need0 = jnp.where(fast_v, _splat(_K), _splat(_K) - _splat(cnt1))

        def bit_body(_, st):
            m, prefix, need_v, endv = st

            @plsc.parallel_loop(0, maxc0, unroll=4, carry=(zero_i, lanes))
            def count_out(i, carry):
                acc, readv = carry
                v = cand_v[pl.ds(i * _L, _L)]
                valid = readv < endv
                hi = (v & m) != zero_i
                acc = acc + jnp.where(jnp.logical_and(valid, hi),
                                      one_i, zero_i)
                return acc, readv + sixteen

            acc, _unused = count_out
            c1_v = plsc.cumsum(acc).at[fifteen].get(
                mode="promise_in_bounds")
            takehi_v = c1_v >= need_v

            def comp_body(i, carry):
                posv, readv = carry
                v = cand_v[pl.ds(i * _L, _L)]
                valid = readv < endv
                hi = (v & m) != zero_i
                keep = jnp.logical_and(valid, hi == takehi_v)
                plsc.store_scatter(cand_v, [posv], v, mask=keep)
                return posv + jnp.where(keep, sixteen, zero_i), readv + sixteen

            (new_endv, _unused2) = lax.fori_loop(0, maxc0, comp_body,
                                                 (lanes, lanes))
            new_need = jnp.where(takehi_v, need_v, need_v - c1_v)
            new_prefix = jnp.where(takehi_v, prefix | m, prefix)
            return (lax.shift_right_logical(m, one_i), new_prefix,
                    new_need, new_endv)

        with jax.named_scope("phase_b"):
            _, prefix, _, _ = lax.fori_loop(
                0, n_iter, bit_body, (m0, prefix0, need0, endv0))

        t_s = jnp.where(fast_v, prefix, prefix ^ min_i)
        b_t = jnp.where(t_s >= zero_i, t_s, min_i - t_s)
        tf = plsc.bitcast(b_t, jnp.float32)
        zf = _splat(0.0, jnp.float32)

        with jax.named_scope("mask"):
            @plsc.parallel_loop(0, _NV_ROW, unroll=8)
            def _mask(i):
                xv = row_v[pl.ds(i * _L, _L)]
                row_v[pl.ds(i * _L, _L)] = jnp.where(xv >= tf, xv, zf)

        with jax.named_scope("dma_out"):
            pltpu.sync_copy(row_v, out_hbm.at[row])
        return 0

    lax.fori_loop(0, _ROWS_PER_W, row_body, 0)


def kernel(x):
    mesh = plsc.VectorSubcoreMesh(core_axis_name="c", subcore_axis_name="s")
    f = functools.partial(
        pl.kernel,
        mesh=mesh,
        out_type=jax.ShapeDtypeStruct((_M, _N), jnp.float32),
        scratch_types=[
            pltpu.VMEM((_N,), jnp.float32),
            pltpu.VMEM((_N,), jnp.int32),
        ],
        compiler_params=pltpu.CompilerParams(needs_layout_passes=False),
    )(_sc_body)
    return f(x)


# speedup vs baseline: 3.1225x; 1.8572x over previous
import functools

import jax
import jax.numpy as jnp
from jax import lax
from jax.experimental import pallas as pl
from jax.experimental.pallas import tpu as pltpu
from jax.experimental.pallas import tpu_sc as plsc

_K = 256
_M = 128
_N = 32768
_L = 16
_NW = 32
_ROWS_PER_W = _M // _NW
_NV_ROW = _N // _L
_HALF = _N
_MININT = -2147483648


def _splat(val, dtype=jnp.int32):
    return jnp.full((_L,), val, dtype=dtype)


def _sc_body(x_hbm, out_hbm, row_v, cand_v):
    wid = lax.axis_index("s") * 2 + lax.axis_index("c")

    lanes = lax.iota(jnp.int32, _L)
    one_i = _splat(1)
    zero_i = _splat(0)
    min_i = _splat(_MININT)
    sixteen = _splat(_L)
    pivf = _splat(2.0, jnp.float32)
    fifteen = _splat(15)

    def compact_fast():
        @plsc.parallel_loop(0, _NV_ROW, unroll=8, carry=lanes)
        def posv_out(i, posv):
            xv = row_v[pl.ds(i * _L, _L)]
            keep = xv >= pivf
            plsc.store_scatter(cand_v, [posv], plsc.bitcast(xv, jnp.int32),
                               mask=keep)
            return posv + jnp.where(keep, sixteen, zero_i)

        return posv_out

    def compact_slow():
        @plsc.parallel_loop(0, _NV_ROW, unroll=8, carry=lanes)
        def posv_out(i, posv):
            xv = row_v[pl.ds(i * _L, _L)]
            b = plsc.bitcast(xv, jnp.int32)
            s = jnp.where(b >= zero_i, b, min_i - b)
            keep = xv < pivf
            plsc.store_scatter(cand_v, [posv], s ^ min_i, mask=keep)
            return posv + jnp.where(keep, sixteen, zero_i)

        return posv_out

    def row_body(r, _):
        row = wid * _ROWS_PER_W + r
        with jax.named_scope("dma_in"):
            pltpu.sync_copy(x_hbm.at[row], row_v)

        with jax.named_scope("phase_a"):
            endv1 = compact_fast()
            cnt1 = jnp.sum(jnp.right_shift(endv1 - lanes, 4))
            fast = cnt1 >= _K
            endv0 = lax.cond(fast, lambda: endv1, compact_slow)

        fast_v = jnp.full((_L,), fast, jnp.bool_)
        maxc0 = jnp.right_shift(jnp.max(endv0 - lanes), 4)
        n_iter = jnp.where(fast, 30, 32)
        m0 = jnp.where(fast_v, _splat(0x20000000), min_i)
        prefix0 = jnp.where(fast_v, _splat(0x40000000), zero_i)
        need0 = jnp.where(fast_v, _splat(_K), _splat(_K) - _splat(cnt1))

        def bit_body(_, st):
            m, prefix, need_v, startv, endv, ws = st
            ws_v = jnp.full((_L,), ws, jnp.int32)
            hicur0 = ws_v + lanes
            locur0 = ws_v + lanes + _splat(_HALF - _L)

            @plsc.parallel_loop(0, maxc0, unroll=4,
                                carry=(hicur0, locur0, startv))
            def part_out(i, carry):
                hicur, locur, readv = carry
                v = plsc.load_gather(cand_v, [readv])
                valid = readv < endv
                hi = (v & m) != zero_i
                hit = jnp.logical_and(valid, hi)
                lot = jnp.logical_and(valid, jnp.logical_not(hi))
                pos = jnp.where(hi, hicur, locur)
                plsc.store_scatter(cand_v, [pos], v, mask=valid)
                hicur = hicur + jnp.where(hit, sixteen, zero_i)
                locur = locur - jnp.where(lot, sixteen, zero_i)
                return hicur, locur, readv + sixteen

            hicur, locur, _unused = part_out
            c1_v = plsc.cumsum(jnp.right_shift(hicur - hicur0, 4)) \
                .at[fifteen].get(mode="promise_in_bounds")
            takehi_v = c1_v >= need_v

            new_start = jnp.where(takehi_v, hicur0, locur + sixteen)
            new_end = jnp.where(takehi_v, hicur,
                                ws_v + lanes + _splat(_HALF))
            new_need = jnp.where(takehi_v, need_v, need_v - c1_v)
            new_prefix = jnp.where(takehi_v, prefix | m, prefix)
            return (lax.shift_right_logical(m, one_i), new_prefix,
                    new_need, new_start, new_end, _HALF - ws)

        with jax.named_scope("phase_b"):
            _, prefix, _, _, _, _ = lax.fori_loop(
                0, n_iter, bit_body,
                (m0, prefix0, need0, lanes, endv0, _HALF))

        t_s = jnp.where(fast_v, prefix, prefix ^ min_i)
        b_t = jnp.where(t_s >= zero_i, t_s, min_i - t_s)
        tf = plsc.bitcast(b_t, jnp.float32)
        zf = _splat(0.0, jnp.float32)

        with jax.named_scope("mask"):
            @plsc.parallel_loop(0, _NV_ROW, unroll=8)
            def _mask(i):
                xv = row_v[pl.ds(i * _L, _L)]
                row_v[pl.ds(i * _L, _L)] = jnp.where(xv >= tf, xv, zf)

        with jax.named_scope("dma_out"):
            pltpu.sync_copy(row_v, out_hbm.at[row])
        return 0

    lax.fori_loop(0, _ROWS_PER_W, row_body, 0)


def kernel(x):
    mesh = plsc.VectorSubcoreMesh(core_axis_name="c", subcore_axis_name="s")
    f = functools.partial(
        pl.kernel,
        mesh=mesh,
        out_type=jax.ShapeDtypeStruct((_M, _N), jnp.float32),
        scratch_types=[
            pltpu.VMEM((_N,), jnp.float32),
            pltpu.VMEM((2 * _N,), jnp.int32),
        ],
        compiler_params=pltpu.CompilerParams(needs_layout_passes=False),
    )(_sc_body)
    return f(x)


# speedup vs baseline: 3.3342x; 1.0678x over previous
import functools

import jax
import jax.numpy as jnp
from jax import lax
from jax.experimental import pallas as pl
from jax.experimental.pallas import tpu as pltpu
from jax.experimental.pallas import tpu_sc as plsc

_K = 256
_M = 128
_N = 32768
_L = 16
_NW = 32
_ROWS_PER_W = _M // _NW
_NV_ROW = _N // _L
_HALF = _N
_MININT = -2147483648


def _splat(val, dtype=jnp.int32):
    return jnp.full((_L,), val, dtype=dtype)


def _sc_body(x_hbm, out_hbm, row_v, cand_v):
    wid = lax.axis_index("s") * 2 + lax.axis_index("c")

    lanes = lax.iota(jnp.int32, _L)
    one_i = _splat(1)
    zero_i = _splat(0)
    min_i = _splat(_MININT)
    sixteen = _splat(_L)
    pivf = _splat(2.25, jnp.float32)
    fifteen = _splat(15)
    neg1_i = _splat(-1)

    def compact_fast():
        @plsc.parallel_loop(0, _NV_ROW, unroll=8,
                            carry=(lanes, neg1_i, zero_i))
        def out(i, carry):
            posv, andv, orv = carry
            xv = row_v[pl.ds(i * _L, _L)]
            key = plsc.bitcast(xv, jnp.int32)
            keep = xv >= pivf
            plsc.store_scatter(cand_v, [posv], key, mask=keep)
            andv = andv & jnp.where(keep, key, neg1_i)
            orv = orv | jnp.where(keep, key, zero_i)
            return posv + jnp.where(keep, sixteen, zero_i), andv, orv

        return out

    def compact_slow():
        @plsc.parallel_loop(0, _NV_ROW, unroll=8,
                            carry=(lanes, neg1_i, zero_i))
        def out(i, carry):
            posv, andv, orv = carry
            xv = row_v[pl.ds(i * _L, _L)]
            b = plsc.bitcast(xv, jnp.int32)
            key = jnp.where(b >= zero_i, b, min_i - b) ^ min_i
            keep = xv < pivf
            plsc.store_scatter(cand_v, [posv], key, mask=keep)
            andv = andv & jnp.where(keep, key, neg1_i)
            orv = orv | jnp.where(keep, key, zero_i)
            return posv + jnp.where(keep, sixteen, zero_i), andv, orv

        return out

    def row_body(r, _):
        row = wid * _ROWS_PER_W + r
        with jax.named_scope("dma_in"):
            pltpu.sync_copy(x_hbm.at[row], row_v)

        with jax.named_scope("phase_a"):
            res1 = compact_fast()
            cnt1 = jnp.sum(jnp.right_shift(res1[0] - lanes, 4))
            fast = cnt1 >= _K
            endv0, andv, orv = lax.cond(fast, lambda: res1, compact_slow)

        fast_v = jnp.full((_L,), fast, jnp.bool_)
        maxc0 = jnp.right_shift(jnp.max(endv0 - lanes), 4)
        def lane_fold(v, op):
            for d in (8, 4, 2, 1):
                sh = v.at[lanes ^ _splat(d)].get(mode="promise_in_bounds")
                v = op(v, sh)
            return v

        and_all = lane_fold(andv, jnp.bitwise_and)
        or_all = lane_fold(orv, jnp.bitwise_or)
        diff = and_all ^ or_all
        dbits = plsc.bitcast(diff.astype(jnp.float32), jnp.int32)
        e_v = jnp.where(diff < zero_i, _splat(31),
                        jnp.maximum(jnp.right_shift(dbits, 23) - _splat(127),
                                    zero_i))
        m0 = lax.shift_left(one_i, e_v)
        lowmask = lax.shift_left(m0, one_i) - one_i
        prefix0 = and_all & ~lowmask
        n_iter = jnp.max(e_v) + 1
        need0 = jnp.where(fast_v, _splat(_K), _splat(_K) - _splat(cnt1))

        def bit_body(_, st):
            m, prefix, need_v, startv, endv, ws = st
            ws_v = jnp.full((_L,), ws, jnp.int32)
            hicur0 = ws_v + lanes
            locur0 = ws_v + lanes + _splat(_HALF - _L)

            @plsc.parallel_loop(0, maxc0, unroll=4,
                                carry=(hicur0, locur0, startv))
            def part_out(i, carry):
                hicur, locur, readv = carry
                v = plsc.load_gather(cand_v, [readv])
                valid = readv < endv
                hi = (v & m) != zero_i
                hit = jnp.logical_and(valid, hi)
                lot = jnp.logical_and(valid, jnp.logical_not(hi))
                pos = jnp.where(hi, hicur, locur)
                plsc.store_scatter(cand_v, [pos], v, mask=valid)
                hicur = hicur + jnp.where(hit, sixteen, zero_i)
                locur = locur - jnp.where(lot, sixteen, zero_i)
                return hicur, locur, readv + sixteen

            hicur, locur, _unused = part_out
            c1_v = plsc.cumsum(jnp.right_shift(hicur - hicur0, 4)) \
                .at[fifteen].get(mode="promise_in_bounds")
            takehi_v = c1_v >= need_v

            new_start = jnp.where(takehi_v, hicur0, locur + sixteen)
            new_end = jnp.where(takehi_v, hicur,
                                ws_v + lanes + _splat(_HALF))
            new_need = jnp.where(takehi_v, need_v, need_v - c1_v)
            new_prefix = jnp.where(takehi_v, prefix | m, prefix)
            return (lax.shift_right_logical(m, one_i), new_prefix,
                    new_need, new_start, new_end, _HALF - ws)

        with jax.named_scope("phase_b"):
            _, prefix, _, _, _, _ = lax.fori_loop(
                0, n_iter, bit_body,
                (m0, prefix0, need0, lanes, endv0, _HALF))

        t_s = jnp.where(fast_v, prefix, prefix ^ min_i)
        b_t = jnp.where(t_s >= zero_i, t_s, min_i - t_s)
        tf = plsc.bitcast(b_t, jnp.float32)
        zf = _splat(0.0, jnp.float32)

        with jax.named_scope("mask"):
            @plsc.parallel_loop(0, _NV_ROW, unroll=8)
            def _mask(i):
                xv = row_v[pl.ds(i * _L, _L)]
                row_v[pl.ds(i * _L, _L)] = jnp.where(xv >= tf, xv, zf)

        with jax.named_scope("dma_out"):
            pltpu.sync_copy(row_v, out_hbm.at[row])
        return 0

    lax.fori_loop(0, _ROWS_PER_W, row_body, 0)


def kernel(x):
    mesh = plsc.VectorSubcoreMesh(core_axis_name="c", subcore_axis_name="s")
    f = functools.partial(
        pl.kernel,
        mesh=mesh,
        out_type=jax.ShapeDtypeStruct((_M, _N), jnp.float32),
        scratch_types=[
            pltpu.VMEM((_N,), jnp.float32),
            pltpu.VMEM((2 * _N,), jnp.int32),
        ],
        compiler_params=pltpu.CompilerParams(needs_layout_passes=False),
    )(_sc_body)
    return f(x)


# speedup vs baseline: 3.4291x; 1.0285x over previous
import functools

import jax
import jax.numpy as jnp
from jax import lax
from jax.experimental import pallas as pl
from jax.experimental.pallas import tpu as pltpu
from jax.experimental.pallas import tpu_sc as plsc

_K = 256
_M = 128
_N = 32768
_L = 16
_NW = 32
_ROWS_PER_W = _M // _NW
_NV_ROW = _N // _L
_HALF = _N
_MININT = -2147483648


def _splat(val, dtype=jnp.int32):
    return jnp.full((_L,), val, dtype=dtype)


def _sc_body(x_hbm, out_hbm, row_v, cand_v):
    wid = lax.axis_index("s") * 2 + lax.axis_index("c")

    lanes = lax.iota(jnp.int32, _L)
    one_i = _splat(1)
    zero_i = _splat(0)
    min_i = _splat(_MININT)
    sixteen = _splat(_L)
    pivf = _splat(2.25, jnp.float32)
    fifteen = _splat(15)
    neg1_i = _splat(-1)

    def compact_fast():
        @plsc.parallel_loop(0, _NV_ROW, unroll=8, carry=(lanes, min_i))
        def out(i, carry):
            posv, maxv = carry
            xv = row_v[pl.ds(i * _L, _L)]
            key = plsc.bitcast(xv, jnp.int32)
            keep = xv >= pivf
            plsc.store_scatter(cand_v, [posv], key, mask=keep)
            return (posv + jnp.where(keep, sixteen, zero_i),
                    jnp.maximum(maxv, key))

        return out

    def compact_slow():
        @plsc.parallel_loop(0, _NV_ROW, unroll=8,
                            carry=(lanes, neg1_i, zero_i))
        def out(i, carry):
            posv, andv, orv = carry
            xv = row_v[pl.ds(i * _L, _L)]
            b = plsc.bitcast(xv, jnp.int32)
            key = jnp.where(b >= zero_i, b, min_i - b) ^ min_i
            keep = xv < pivf
            plsc.store_scatter(cand_v, [posv], key, mask=keep)
            andv = andv & jnp.where(keep, key, neg1_i)
            orv = orv | jnp.where(keep, key, zero_i)
            return posv + jnp.where(keep, sixteen, zero_i), andv, orv

        return out

    def row_body(r, _):
        row = wid * _ROWS_PER_W + r
        with jax.named_scope("dma_in"):
            pltpu.sync_copy(x_hbm.at[row], row_v)

        def lane_fold(v, op):
            for d in (8, 4, 2, 1):
                sh = v.at[lanes ^ _splat(d)].get(mode="promise_in_bounds")
                v = op(v, sh)
            return v

        with jax.named_scope("phase_a"):
            endv1, maxv = compact_fast()
            cnt1 = jnp.sum(jnp.right_shift(endv1 - lanes, 4))
            fast = cnt1 >= _K
            endv0, lo_all, hi_all = lax.cond(
                fast,
                lambda: (endv1, plsc.bitcast(pivf, jnp.int32),
                         lane_fold(maxv, jnp.maximum)),
                lambda: (lambda r: (r[0], lane_fold(r[1], jnp.bitwise_and),
                                    lane_fold(r[2], jnp.bitwise_or)))(
                    compact_slow()),
            )

        fast_v = jnp.full((_L,), fast, jnp.bool_)
        maxc0 = jnp.right_shift(jnp.max(endv0 - lanes), 4)
        diff = lo_all ^ hi_all
        dbits = plsc.bitcast(diff.astype(jnp.float32), jnp.int32)
        e_v = jnp.where(diff < zero_i, _splat(31),
                        jnp.maximum(jnp.right_shift(dbits, 23) - _splat(127),
                                    zero_i))
        m0 = lax.shift_left(one_i, e_v)
        lowmask = lax.shift_left(m0, one_i) - one_i
        prefix0 = lo_all & ~lowmask
        n_iter = jnp.max(e_v) + 1
        need0 = jnp.where(fast_v, _splat(_K), _splat(_K) - _splat(cnt1))

        def bit_body(_, st):
            m, prefix, need_v, startv, endv, ws = st
            ws_v = jnp.full((_L,), ws, jnp.int32)
            hicur0 = ws_v + lanes
            locur0 = ws_v + lanes + _splat(_HALF - _L)

            @plsc.parallel_loop(0, maxc0, unroll=4,
                                carry=(hicur0, locur0, startv))
            def part_out(i, carry):
                hicur, locur, readv = carry
                v = plsc.load_gather(cand_v, [readv])
                valid = readv < endv
                hi = (v & m) != zero_i
                hit = jnp.logical_and(valid, hi)
                lot = jnp.logical_and(valid, jnp.logical_not(hi))
                pos = jnp.where(hi, hicur, locur)
                plsc.store_scatter(cand_v, [pos], v, mask=valid)
                hicur = hicur + jnp.where(hit, sixteen, zero_i)
                locur = locur - jnp.where(lot, sixteen, zero_i)
                return hicur, locur, readv + sixteen

            hicur, locur, _unused = part_out
            c1_v = plsc.cumsum(jnp.right_shift(hicur - hicur0, 4)) \
                .at[fifteen].get(mode="promise_in_bounds")
            takehi_v = c1_v >= need_v

            new_start = jnp.where(takehi_v, hicur0, locur + sixteen)
            new_end = jnp.where(takehi_v, hicur,
                                ws_v + lanes + _splat(_HALF))
            new_need = jnp.where(takehi_v, need_v, need_v - c1_v)
            new_prefix = jnp.where(takehi_v, prefix | m, prefix)
            return (lax.shift_right_logical(m, one_i), new_prefix,
                    new_need, new_start, new_end, _HALF - ws)

        with jax.named_scope("phase_b"):
            _, prefix, _, _, _, _ = lax.fori_loop(
                0, n_iter, bit_body,
                (m0, prefix0, need0, lanes, endv0, _HALF))

        t_s = jnp.where(fast_v, prefix, prefix ^ min_i)
        b_t = jnp.where(t_s >= zero_i, t_s, min_i - t_s)
        tf = plsc.bitcast(b_t, jnp.float32)
        zf = _splat(0.0, jnp.float32)

        with jax.named_scope("mask"):
            @plsc.parallel_loop(0, _NV_ROW, unroll=8)
            def _mask(i):
                xv = row_v[pl.ds(i * _L, _L)]
                row_v[pl.ds(i * _L, _L)] = jnp.where(xv >= tf, xv, zf)

        with jax.named_scope("dma_out"):
            pltpu.sync_copy(row_v, out_hbm.at[row])
        return 0

    lax.fori_loop(0, _ROWS_PER_W, row_body, 0)


def kernel(x):
    mesh = plsc.VectorSubcoreMesh(core_axis_name="c", subcore_axis_name="s")
    f = functools.partial(
        pl.kernel,
        mesh=mesh,
        out_type=jax.ShapeDtypeStruct((_M, _N), jnp.float32),
        scratch_types=[
            pltpu.VMEM((_N,), jnp.float32),
            pltpu.VMEM((2 * _N,), jnp.int32),
        ],
        compiler_params=pltpu.CompilerParams(needs_layout_passes=False),
    )(_sc_body)
    return f(x)
